# baseline (speedup 1.0000x reference)
import functools

import jax
import jax.numpy as jnp
from jax import lax
from jax.experimental import pallas as pl
from jax.experimental.pallas import tpu as pltpu
from jax.experimental.pallas import tpu_sc as plsc

_CH = 128


@functools.lru_cache(maxsize=None)
def _make_sc_kernel(B, NS, VS):
    info = plsc.get_sparse_core_info()
    NC, NSUB, L = info.num_cores, info.num_subcores, info.num_lanes
    NW = NC * NSUB
    bpw = B // NW
    NCH = bpw // _CH
    NSL = bpw // L
    assert B % NW == 0 and bpw % _CH == 0 and bpw % L == 0

    mesh = plsc.VectorSubcoreMesh(core_axis_name="c", subcore_axis_name="s")

    @functools.partial(
        pl.kernel,
        mesh=mesh,
        compiler_params=pltpu.CompilerParams(use_tc_tiling_on_sc=False),
        out_type=jax.ShapeDtypeStruct((B,), jnp.float32),
        scratch_types=[
            pltpu.VMEM((bpw,), jnp.float32),
            pltpu.VMEM((bpw,), jnp.int32),
            pltpu.VMEM((bpw,), jnp.int32),
            pltpu.VMEM((bpw,), jnp.float32),
            pltpu.VMEM((bpw,), jnp.float32),
            pltpu.SemaphoreType.DMA,
            pltpu.SemaphoreType.DMA,
        ],
    )
    def k(idx_hbm, proj_hbm, dvec_hbm, out_hbm,
          acc_v, idx0, idx1, g0, g1, sem0, sem1):
        wid = lax.axis_index("s") * NC + lax.axis_index("c")
        base = wid * bpw

        idxs = (idx0, idx1)
        gs = (g0, g1)
        sems = (sem0, sem1)

        pltpu.sync_copy(dvec_hbm.at[pl.ds(base, bpw)], acc_v)

        def prep_and_fire(f):
            par = f % 2
            ib, gb, sm = idxs[par], gs[par], sems[par]
            pltpu.sync_copy(idx_hbm.at[f, pl.ds(base, bpw)], ib)
            off = jnp.int32(f * VS)

            def add_off(i, carry):
                sl = pl.ds(i * L, L)
                ib[sl] = ib[sl] + off
                return carry

            lax.fori_loop(0, NSL, add_off, 0)
            return [
                pltpu.async_copy(proj_hbm.at[ib.at[pl.ds(j * _CH, _CH)]],
                                 gb.at[pl.ds(j * _CH, _CH)], sm)
                for j in range(NCH)
            ]

        def accum(f):
            gb = gs[f % 2]

            def body(i, carry):
                sl = pl.ds(i * L, L)
                acc_v[sl] = acc_v[sl] + gb[sl]
                return carry

            lax.fori_loop(0, NSL, body, 0)

        handles = prep_and_fire(0)
        for f in range(NS):
            nxt = prep_and_fire(f + 1) if f + 1 < NS else None
            for h in handles:
                h.wait()
            accum(f)
            handles = nxt

        pltpu.sync_copy(acc_v, out_hbm.at[pl.ds(base, bpw)])

    return k


def _proj_tc(NS, V, E, VS):
    def body(t_ref, w_ref, o_ref):
        w = w_ref[pl.program_id(0), :]
        vals = jnp.sum(t_ref[0] * w[:, None], axis=0)
        vals = jnp.concatenate(
            [vals, jnp.zeros((VS - V,), jnp.float32)])
        o_ref[...] = vals.reshape(1, VS // 128, 128)

    return pl.pallas_call(
        body,
        grid=(NS,),
        in_specs=[
            pl.BlockSpec((1, E, V), lambda f: (f, 0, 0)),
            pl.BlockSpec((NS, E), lambda f: (0, 0)),
        ],
        out_specs=pl.BlockSpec((1, VS // 128, 128), lambda f: (f, 0, 0)),
        out_shape=jax.ShapeDtypeStruct((NS, VS // 128, 128), jnp.float32),
    )


def _dense_tc(B, ND, blk):
    def body(d_ref, wd_ref, bd_ref, bs_ref, o_ref):
        acc = jnp.full((blk,), bd_ref[0, 0] + bs_ref[0, 0], jnp.float32)
        for d in range(ND):
            acc = acc + d_ref[d, :] * wd_ref[d, 0]
        o_ref[...] = acc

    return pl.pallas_call(
        body,
        grid=(B // blk,),
        in_specs=[
            pl.BlockSpec((ND, blk), lambda i: (0, i)),
            pl.BlockSpec((ND, 1), lambda i: (0, 0)),
            pl.BlockSpec((1, 1), lambda i: (0, 0)),
            pl.BlockSpec((1, 1), lambda i: (0, 0)),
        ],
        out_specs=pl.BlockSpec((blk,), lambda i: (i,)),
        out_shape=jax.ShapeDtypeStruct((B,), jnp.float32),
    )


def kernel(dense_features, sparse_features, tables, W_dense, b_dense,
           W_sparse, b_sparse):
    B, ND = dense_features.shape
    NS, V, E = tables.shape

    VS = -(-V // 1024) * 1024

    tables_r = jnp.transpose(tables, (0, 2, 1))
    w2 = W_sparse.reshape(NS, E)
    proj = _proj_tc(NS, V, E, VS)(tables_r, w2)

    dense_t = dense_features.T
    dvec = _dense_tc(B, ND, 2048)(dense_t, W_dense, b_dense.reshape(1, 1),
                                  b_sparse.reshape(1, 1))

    idx_t = sparse_features.T.astype(jnp.int32)
    out = _make_sc_kernel(B, NS, VS)(idx_t, proj.reshape(NS * VS), dvec)
    return out
